# baseline (speedup 1.0000x reference)
import jax
import jax.numpy as jnp
from jax.experimental import pallas as pl

_ROWS = 16384
_COLS = 128
_BLOCK_ROWS = 16384


def _copy_set_kernel(x_ref, o_ref):
    o_ref[...] = x_ref[...]

    @pl.when(pl.program_id(0) == 0)
    def _():
        col = jax.lax.broadcasted_iota(jnp.int32, (1, _COLS), 1)
        o_ref[0:1, :] = jnp.where(col == 0, 3.0, x_ref[0:1, :])


def kernel(x):
    grid = (_ROWS // _BLOCK_ROWS,)
    return pl.pallas_call(
        _copy_set_kernel,
        grid=grid,
        in_specs=[pl.BlockSpec((_BLOCK_ROWS, _COLS), lambda i: (i, 0))],
        out_specs=pl.BlockSpec((_BLOCK_ROWS, _COLS), lambda i: (i, 0)),
        out_shape=jax.ShapeDtypeStruct((_ROWS, _COLS), jnp.float32),
    )(x)


# speedup vs baseline: 1.2287x; 1.2287x over previous
import jax
import jax.numpy as jnp
from jax.experimental import pallas as pl
from jax.experimental.pallas import tpu as pltpu

_ROWS = 16384
_COLS = 128
_CHUNK = 4096
_NSLOTS = 4
_NCHUNKS = _ROWS // _CHUNK


def _copy_set_kernel(x_hbm, o_hbm, scratch, in_sems, out_sems):
    def in_copy(c):
        slot = c % _NSLOTS
        return pltpu.make_async_copy(
            x_hbm.at[pl.ds(c * _CHUNK, _CHUNK), :],
            scratch.at[slot], in_sems.at[slot])

    def out_copy(c):
        slot = c % _NSLOTS
        return pltpu.make_async_copy(
            scratch.at[slot],
            o_hbm.at[pl.ds(c * _CHUNK, _CHUNK), :], out_sems.at[slot])

    for c in range(min(_NSLOTS, _NCHUNKS)):
        in_copy(c).start()
    for c in range(_NCHUNKS):
        in_copy(c).wait()
        if c == 0:
            col = jax.lax.broadcasted_iota(jnp.int32, (1, _COLS), 1)
            scratch[0, 0:1, :] = jnp.where(col == 0, 3.0, scratch[0, 0:1, :])
        out_copy(c).start()
        nxt = c + _NSLOTS
        if nxt < _NCHUNKS:
            out_copy(c).wait()
            in_copy(nxt).start()
    for c in range(max(_NCHUNKS - _NSLOTS, 0), _NCHUNKS):
        out_copy(c).wait()


def kernel(x):
    return pl.pallas_call(
        _copy_set_kernel,
        in_specs=[pl.BlockSpec(memory_space=pl.ANY)],
        out_specs=pl.BlockSpec(memory_space=pl.ANY),
        out_shape=jax.ShapeDtypeStruct((_ROWS, _COLS), jnp.float32),
        scratch_shapes=[
            pltpu.VMEM((_NSLOTS, _CHUNK, _COLS), jnp.float32),
            pltpu.SemaphoreType.DMA((_NSLOTS,)),
            pltpu.SemaphoreType.DMA((_NSLOTS,)),
        ],
    )(x)


# speedup vs baseline: 1.2287x; 1.0000x over previous
import jax
import jax.numpy as jnp
from jax.experimental import pallas as pl
from jax.experimental.pallas import tpu as pltpu

_ROWS = 16384
_COLS = 128
_CHUNK = 2048
_NSLOTS = 8
_NCHUNKS = _ROWS // _CHUNK


def _copy_set_kernel(x_hbm, o_hbm, scratch, in_sems, out_sems):
    def in_copy(c):
        slot = c % _NSLOTS
        return pltpu.make_async_copy(
            x_hbm.at[pl.ds(c * _CHUNK, _CHUNK), :],
            scratch.at[slot], in_sems.at[slot])

    def out_copy(c):
        slot = c % _NSLOTS
        return pltpu.make_async_copy(
            scratch.at[slot],
            o_hbm.at[pl.ds(c * _CHUNK, _CHUNK), :], out_sems.at[slot])

    for c in range(min(_NSLOTS, _NCHUNKS)):
        in_copy(c).start()
    for c in range(_NCHUNKS):
        in_copy(c).wait()
        if c == 0:
            col = jax.lax.broadcasted_iota(jnp.int32, (1, _COLS), 1)
            scratch[0, 0:1, :] = jnp.where(col == 0, 3.0, scratch[0, 0:1, :])
        out_copy(c).start()
        nxt = c + _NSLOTS
        if nxt < _NCHUNKS:
            out_copy(c).wait()
            in_copy(nxt).start()
    for c in range(max(_NCHUNKS - _NSLOTS, 0), _NCHUNKS):
        out_copy(c).wait()


def kernel(x):
    return pl.pallas_call(
        _copy_set_kernel,
        in_specs=[pl.BlockSpec(memory_space=pl.ANY)],
        out_specs=pl.BlockSpec(memory_space=pl.ANY),
        out_shape=jax.ShapeDtypeStruct((_ROWS, _COLS), jnp.float32),
        scratch_shapes=[
            pltpu.VMEM((_NSLOTS, _CHUNK, _COLS), jnp.float32),
            pltpu.SemaphoreType.DMA((_NSLOTS,)),
            pltpu.SemaphoreType.DMA((_NSLOTS,)),
        ],
    )(x)
